# baseline (speedup 1.0000x reference)
import functools
import jax
import jax.numpy as jnp
from jax import lax
from jax.experimental import pallas as pl
from jax.experimental.pallas import tpu as pltpu
from jax.experimental.pallas import tpu_sc as plsc

N = 10000
NP = 10240
E = 320000
D = 128
H = 32
B = 64
T = 50

NC = 2
NS = 16
NW = NC * NS
CH = 128
NCHT = 80
NCHUNK = NW * NCHT
EPAD = NCHUNK * CH - E
NODES_PER_TILE = NP // NS
NBUF = 5

_mesh = plsc.VectorSubcoreMesh(core_axis_name="c", subcore_axis_name="s")


def _prep0_body(x_ref, w_ref, asrc_ref, adst_ref, we_ref, ae_ref,
                wx_ref, ss_ref, sd_ref, cv_ref):
    h = x_ref[...]
    wx = jnp.dot(h, w_ref[...], preferred_element_type=jnp.float32)
    wx_ref[...] = wx
    ss_ref[...] = jnp.sum(wx * asrc_ref[...], axis=1)
    sd_ref[...] = jnp.sum(wx * adst_ref[...], axis=1)
    c = jnp.sum(we_ref[...] * ae_ref[...])
    cv_ref[...] = jnp.full((16,), c, jnp.float32)


def _prep_next_body(acc_ref, den_ref, b_ref, w_ref, asrc_ref, adst_ref,
                    we_ref, ae_ref, wx_ref, ss_ref, sd_ref, cv_ref, *, relu):
    den = den_ref[0, :] + den_ref[1, :]
    hnode = (acc_ref[0, :, :] + acc_ref[1, :, :]) / (den[:, None] + 1e-16)
    hnode = hnode + b_ref[...]
    if relu:
        hnode = jnp.maximum(hnode, 0.0)
    wx = jnp.dot(hnode, w_ref[...], preferred_element_type=jnp.float32)
    wx_ref[...] = wx
    ss_ref[...] = jnp.sum(wx * asrc_ref[...], axis=1)
    sd_ref[...] = jnp.sum(wx * adst_ref[...], axis=1)
    c = jnp.sum(we_ref[...] * ae_ref[...])
    cv_ref[...] = jnp.full((16,), c, jnp.float32)


_BN = NP
_GRID = NP // _BN


def _prep0(xpad, w, asrc, adst, we, ae):
    rep = lambda shp: pl.BlockSpec(shp, lambda i: tuple(0 for _ in shp))
    return pl.pallas_call(
        _prep0_body,
        grid=(_GRID,),
        in_specs=[
            pl.BlockSpec((_BN, D), lambda i: (i, 0)),
            rep((D, H)), rep((1, H)), rep((1, H)), rep((1, H)), rep((1, H)),
        ],
        out_specs=[
            pl.BlockSpec((_BN, H), lambda i: (i, 0)),
            pl.BlockSpec((_BN,), lambda i: (i,)),
            pl.BlockSpec((_BN,), lambda i: (i,)),
            pl.BlockSpec((16,), lambda i: (0,)),
        ],
        out_shape=[
            jax.ShapeDtypeStruct((NP, H), jnp.float32),
            jax.ShapeDtypeStruct((NP,), jnp.float32),
            jax.ShapeDtypeStruct((NP,), jnp.float32),
            jax.ShapeDtypeStruct((16,), jnp.float32),
        ],
    )(xpad, w, asrc, adst, we, ae)


def _prep_next(acc, den, b, w, asrc, adst, we, ae, relu):
    rep = lambda shp: pl.BlockSpec(shp, lambda i: tuple(0 for _ in shp))
    return pl.pallas_call(
        functools.partial(_prep_next_body, relu=relu),
        grid=(_GRID,),
        in_specs=[
            pl.BlockSpec((2, _BN, H), lambda i: (0, i, 0)),
            pl.BlockSpec((2, _BN), lambda i: (0, i)),
            rep((1, H)),
            pl.BlockSpec((H, H), lambda i: (0, 0)),
            rep((1, H)), rep((1, H)), rep((1, H)), rep((1, H)),
        ],
        out_specs=[
            pl.BlockSpec((_BN, H), lambda i: (i, 0)),
            pl.BlockSpec((_BN,), lambda i: (i,)),
            pl.BlockSpec((_BN,), lambda i: (i,)),
            pl.BlockSpec((16,), lambda i: (0,)),
        ],
        out_shape=[
            jax.ShapeDtypeStruct((NP, H), jnp.float32),
            jax.ShapeDtypeStruct((NP,), jnp.float32),
            jax.ShapeDtypeStruct((NP,), jnp.float32),
            jax.ShapeDtypeStruct((16,), jnp.float32),
        ],
    )(acc, den, b, w, asrc, adst, we, ae)


def _gat_edge_body(src_hbm, dst_hbm, ea_hbm, wx_hbm, ss_hbm, sd_hbm, cv_hbm,
                   den_out, acc_out, *rest):
    bufs = []
    for b in range(NBUF):
        bufs.append(tuple(rest[b * 5:(b + 1) * 5]))
    (srcres, dstres, eares, cvv, stage, dstage,
     den_sp, acc_sp) = rest[NBUF * 5:NBUF * 5 + 8]
    semL = rest[NBUF * 5 + 8]
    sems = rest[NBUF * 5 + 9:]
    for b in range(NBUF):
        bufs[b] = bufs[b] + tuple(sems[b * 2:(b + 1) * 2])

    cid = lax.axis_index("c")
    sid = lax.axis_index("s")
    wid = sid * NC + cid

    pltpu.async_copy(src_hbm.at[wid], srcres, semL)
    pltpu.async_copy(dst_hbm.at[wid], dstres, semL)
    pltpu.async_copy(ea_hbm.at[wid], eares, semL)
    pltpu.sync_copy(cv_hbm, cvv)

    z16 = jnp.zeros((16,), jnp.float32)

    def _zrow(r, _):
        stage[r, pl.ds(0, 16)] = z16
        stage[r, pl.ds(16, 16)] = z16
        return 0
    lax.fori_loop(0, NODES_PER_TILE, _zrow, 0)

    def _zden(j, _):
        dstage[pl.ds(j * 16, 16)] = z16
        return 0
    lax.fori_loop(0, NODES_PER_TILE // 16, _zden, 0)

    nbase = sid * NODES_PER_TILE
    pltpu.sync_copy(stage, acc_sp.at[pl.ds(nbase, NODES_PER_TILE), :])
    pltpu.sync_copy(dstage, den_sp.at[pl.ds(nbase, NODES_PER_TILE)])
    plsc.subcore_barrier()

    pltpu.make_async_copy(src_hbm.at[0], srcres, semL).wait()
    pltpu.make_async_copy(dst_hbm.at[0], dstres, semL).wait()
    pltpu.make_async_copy(ea_hbm.at[0], eares, semL).wait()

    def issue_g(buf, k):
        _, ssg, sdg, rows, _, semG, _ = buf
        pltpu.async_copy(ss_hbm.at[srcres.at[k]], ssg, semG)
        pltpu.async_copy(sd_hbm.at[dstres.at[k]], sdg, semG)
        pltpu.async_copy(wx_hbm.at[srcres.at[k]], rows, semG)

    def wait_g(buf):
        _, ssg, sdg, rows, _, semG, _ = buf
        pltpu.make_async_copy(ss_hbm.at[srcres.at[0]], ssg, semG).wait()
        pltpu.make_async_copy(sd_hbm.at[dstres.at[0]], sdg, semG).wait()
        pltpu.make_async_copy(wx_hbm.at[srcres.at[0]], rows, semG).wait()

    def compute_core(buf, dst_ld, ea_ld):
        dstv_s, ssg, sdg, rows, av, _, _ = buf
        cvec = cvv[...]
        for j in range(CH // 16):
            dstv_s[pl.ds(j * 16, 16)] = dst_ld(j)
            s = ssg[pl.ds(j * 16, 16)] + sdg[pl.ds(j * 16, 16)] \
                + cvec * ea_ld(j)
            s = jnp.where(s > 0.0, s, 0.2 * s)
            av[pl.ds(j * 16, 16)] = jnp.exp(s)
        for j in range(CH // 16):
            a16 = av[pl.ds(j * 16, 16)]
            for i in range(16):
                e = j * 16 + i
                a = a16[i]
                rows[e, pl.ds(0, 16)] = rows[e, pl.ds(0, 16)] * a
                rows[e, pl.ds(16, 16)] = rows[e, pl.ds(16, 16)] * a

    def compute(buf, k):
        compute_core(buf,
                     lambda j: dstres[k, pl.ds(j * 16, 16)],
                     lambda j: eares[k, pl.ds(j * 16, 16)])

    def issue_s(buf):
        dstv_s, _, _, rows, av, _, semS = buf
        pltpu.async_copy(av, den_sp.at[dstv_s], semS, add=True)
        pltpu.async_copy(rows, acc_sp.at[dstv_s], semS, add=True)

    def wait_s(buf):
        dstv_s, _, _, rows, av, _, semS = buf
        pltpu.make_async_copy(av, den_sp.at[dstv_s], semS).wait()
        pltpu.make_async_copy(rows, acc_sp.at[dstv_s], semS).wait()

    for b in range(NBUF - 1):
        issue_g(bufs[b], b)

    def _body(i, _):
        for b in range(NBUF):
            k = i * NBUF + b
            buf = bufs[b]
            b3 = (b + NBUF - 1) % NBUF

            wait_g(buf)
            compute(buf, k)
            issue_s(buf)

            @pl.when(jnp.logical_and(k >= 1, k + NBUF - 1 < NCHT))
            def _():
                wait_s(bufs[b3])

            @pl.when(k + NBUF - 1 < NCHT)
            def _():
                issue_g(bufs[b3], k + NBUF - 1)
        return 0

    lax.fori_loop(0, NCHT // NBUF, _body, 0)
    for b in range(NBUF):
        wait_s(bufs[b])

    plsc.subcore_barrier()

    pltpu.sync_copy(acc_sp.at[pl.ds(nbase, NODES_PER_TILE), :], stage)
    pltpu.sync_copy(stage, acc_out.at[cid, pl.ds(nbase, NODES_PER_TILE), :])
    pltpu.sync_copy(den_sp.at[pl.ds(nbase, NODES_PER_TILE)], dstage)
    pltpu.sync_copy(dstage, den_out.at[cid, pl.ds(nbase, NODES_PER_TILE)])


_gat_edge = pl.kernel(
    _gat_edge_body,
    out_type=[
        jax.ShapeDtypeStruct((NC, NP), jnp.float32),
        jax.ShapeDtypeStruct((NC, NP, H), jnp.float32),
    ],
    mesh=_mesh,
    compiler_params=pltpu.CompilerParams(use_tc_tiling_on_sc=False),
    scratch_types=(
        [pltpu.VMEM((CH,), jnp.int32),
         pltpu.VMEM((CH,), jnp.float32),
         pltpu.VMEM((CH,), jnp.float32),
         pltpu.VMEM((CH, H), jnp.float32),
         pltpu.VMEM((CH,), jnp.float32)]
        * NBUF
        + [pltpu.VMEM((NCHT, CH), jnp.int32),
           pltpu.VMEM((NCHT, CH), jnp.int32),
           pltpu.VMEM((NCHT, CH), jnp.float32),
           pltpu.VMEM((16,), jnp.float32),
           pltpu.VMEM((NODES_PER_TILE, H), jnp.float32),
           pltpu.VMEM((NODES_PER_TILE,), jnp.float32),
           pltpu.VMEM_SHARED((NP,), jnp.float32),
           pltpu.VMEM_SHARED((NP, H), jnp.float32)]
        + [pltpu.SemaphoreType.DMA] * (1 + 2 * NBUF)
    ),
)


def _gru_body(acc_ref, den_ref, b2_ref, cs_ref, tgt_ref,
              wi0r_ref, wi0z_ref, wi0n_ref, wh0r_ref, wh0z_ref, wh0n_ref,
              wi1r_ref, wi1z_ref, wi1n_ref, wh1r_ref, wh1z_ref, wh1n_ref,
              bi0r_ref, bi0z_ref, bi0n_ref, bh0r_ref, bh0z_ref, bh0n_ref,
              bi1r_ref, bi1z_ref, bi1n_ref, bh1r_ref, bh1z_ref, bh1n_ref,
              wout_ref, bout_ref,
              y_ref, gir_s, giz_s, gin_s):
    tgt = tgt_ref[...]
    dot = lambda a, b: jnp.dot(a, b, preferred_element_type=jnp.float32)
    gir_s[...] = dot(tgt, wi0r_ref[...]) + bi0r_ref[...]
    giz_s[...] = dot(tgt, wi0z_ref[...]) + bi0z_ref[...]
    gin_s[...] = dot(tgt, wi0n_ref[...]) + bi0n_ref[...]

    den = den_ref[0, :] + den_ref[1, :]
    h2 = (acc_ref[0, :, :] + acc_ref[1, :, :]) / (den[:, None] + 1e-16) \
        + b2_ref[...]
    node_ids = jax.lax.broadcasted_iota(jnp.int32, (B, NP), 1)
    onehot = (node_ids == cs_ref[...]).astype(jnp.float32)
    emb = dot(onehot, h2)

    def step(t, carry):
        h0, h1 = carry
        r0 = jax.nn.sigmoid(gir_s[pl.ds(t * B, B), :]
                            + dot(h0, wh0r_ref[...]) + bh0r_ref[...])
        z0 = jax.nn.sigmoid(giz_s[pl.ds(t * B, B), :]
                            + dot(h0, wh0z_ref[...]) + bh0z_ref[...])
        n0 = jnp.tanh(gin_s[pl.ds(t * B, B), :]
                      + r0 * (dot(h0, wh0n_ref[...]) + bh0n_ref[...]))
        h0n = (1.0 - z0) * n0 + z0 * h0

        r1 = jax.nn.sigmoid(dot(h0n, wi1r_ref[...]) + bi1r_ref[...]
                            + dot(h1, wh1r_ref[...]) + bh1r_ref[...])
        z1 = jax.nn.sigmoid(dot(h0n, wi1z_ref[...]) + bi1z_ref[...]
                            + dot(h1, wh1z_ref[...]) + bh1z_ref[...])
        n1 = jnp.tanh(dot(h0n, wi1n_ref[...]) + bi1n_ref[...]
                      + r1 * (dot(h1, wh1n_ref[...]) + bh1n_ref[...]))
        h1n = (1.0 - z1) * n1 + z1 * h1

        y_ref[pl.ds(t * B, B), :] = dot(h1n, wout_ref[...]) + bout_ref[...]
        return (h0n, h1n)

    lax.fori_loop(0, T, step, (emb, emb))


def _gru(acc, den, b2, cs2d, tgt_tmaj, *args):
    return pl.pallas_call(
        _gru_body,
        out_shape=jax.ShapeDtypeStruct((T * B, D), jnp.float32),
        scratch_shapes=[
            pltpu.VMEM((T * B, H), jnp.float32),
            pltpu.VMEM((T * B, H), jnp.float32),
            pltpu.VMEM((T * B, H), jnp.float32),
        ],
    )(acc, den, b2, cs2d, tgt_tmaj, *args)


def kernel(x, edge_index, edge_attr, cur_states, tgt_seq,
           gat0_W, gat0_We, gat0_asrc, gat0_adst, gat0_ae, gat0_b,
           gat1_W, gat1_We, gat1_asrc, gat1_adst, gat1_ae, gat1_b,
           gat2_W, gat2_We, gat2_asrc, gat2_adst, gat2_ae, gat2_b,
           W_ih0, W_hh0, b_ih0, b_hh0,
           W_ih1, W_hh1, b_ih1, b_hh1,
           W_out, b_out):
    xpad = jnp.pad(x, ((0, NP - N), (0, 0)))
    pad_idx = N + (jnp.arange(EPAD, dtype=jnp.int32) % (NP - N))
    src2d = jnp.concatenate([edge_index[0], pad_idx]).reshape(NW, NCHT, CH)
    dst2d = jnp.concatenate([edge_index[1], pad_idx]).reshape(NW, NCHT, CH)
    ea2d = jnp.concatenate([edge_attr[:, 0],
                            jnp.zeros((EPAD,), jnp.float32)]).reshape(NW, NCHT, CH)

    r1 = lambda v: v.reshape(1, -1)

    wx, ss, sd, cv = _prep0(xpad, gat0_W, r1(gat0_asrc), r1(gat0_adst),
                            gat0_We, r1(gat0_ae))
    den, acc = _gat_edge(src2d, dst2d, ea2d, wx, ss, sd, cv)

    wx, ss, sd, cv = _prep_next(acc, den, r1(gat0_b), gat1_W, r1(gat1_asrc),
                                r1(gat1_adst), gat1_We, r1(gat1_ae), relu=True)
    den, acc = _gat_edge(src2d, dst2d, ea2d, wx, ss, sd, cv)

    wx, ss, sd, cv = _prep_next(acc, den, r1(gat1_b), gat2_W, r1(gat2_asrc),
                                r1(gat2_adst), gat2_We, r1(gat2_ae), relu=True)
    den, acc = _gat_edge(src2d, dst2d, ea2d, wx, ss, sd, cv)

    tgt_tmaj = jnp.swapaxes(tgt_seq, 0, 1).reshape(T * B, D)
    wi0 = W_ih0.T
    wh0 = W_hh0.T
    wi1 = W_ih1.T
    wh1 = W_hh1.T
    y = _gru(acc, den, r1(gat2_b), cur_states.reshape(B, 1), tgt_tmaj,
             wi0[:, 0:H], wi0[:, H:2 * H], wi0[:, 2 * H:3 * H],
             wh0[:, 0:H], wh0[:, H:2 * H], wh0[:, 2 * H:3 * H],
             wi1[:, 0:H], wi1[:, H:2 * H], wi1[:, 2 * H:3 * H],
             wh1[:, 0:H], wh1[:, H:2 * H], wh1[:, 2 * H:3 * H],
             r1(b_ih0[0:H]), r1(b_ih0[H:2 * H]), r1(b_ih0[2 * H:3 * H]),
             r1(b_hh0[0:H]), r1(b_hh0[H:2 * H]), r1(b_hh0[2 * H:3 * H]),
             r1(b_ih1[0:H]), r1(b_ih1[H:2 * H]), r1(b_ih1[2 * H:3 * H]),
             r1(b_hh1[0:H]), r1(b_hh1[H:2 * H]), r1(b_hh1[2 * H:3 * H]),
             W_out.T, r1(b_out))
    y = y.reshape(T, B, D).swapaxes(0, 1)
    return y


# speedup vs baseline: 1.0935x; 1.0935x over previous
import functools
import jax
import jax.numpy as jnp
from jax import lax
from jax.experimental import pallas as pl
from jax.experimental.pallas import tpu as pltpu
from jax.experimental.pallas import tpu_sc as plsc

N = 10000
NP = 10240
E = 320000
D = 128
H = 32
B = 64
T = 50

NC = 2
NS = 16
NW = NC * NS
CH = 128
NCHT = 80
NCHUNK = NW * NCHT
EPAD = NCHUNK * CH - E
NODES_PER_TILE = NP // NS
NBUF = 5

_mesh = plsc.VectorSubcoreMesh(core_axis_name="c", subcore_axis_name="s")


def _prep0_body(x_ref, w_ref, asrc_ref, adst_ref, we_ref, ae_ref,
                wx_ref, ss_ref, sd_ref, cv_ref):
    h = x_ref[...]
    wx = jnp.dot(h, w_ref[...], preferred_element_type=jnp.float32)
    wx_ref[...] = wx
    ss_ref[...] = jnp.sum(wx * asrc_ref[...], axis=1)
    sd_ref[...] = jnp.sum(wx * adst_ref[...], axis=1)
    c = jnp.sum(we_ref[...] * ae_ref[...])
    cv_ref[...] = jnp.full((16,), c, jnp.float32)


def _prep_next_body(acc_ref, den_ref, b_ref, w_ref, asrc_ref, adst_ref,
                    we_ref, ae_ref, wx_ref, ss_ref, sd_ref, cv_ref, *, relu):
    den = den_ref[0, :] + den_ref[1, :]
    hnode = (acc_ref[0, :, :] + acc_ref[1, :, :]) / (den[:, None] + 1e-16)
    hnode = hnode + b_ref[...]
    if relu:
        hnode = jnp.maximum(hnode, 0.0)
    wx = jnp.dot(hnode, w_ref[...], preferred_element_type=jnp.float32)
    wx_ref[...] = wx
    ss_ref[...] = jnp.sum(wx * asrc_ref[...], axis=1)
    sd_ref[...] = jnp.sum(wx * adst_ref[...], axis=1)
    c = jnp.sum(we_ref[...] * ae_ref[...])
    cv_ref[...] = jnp.full((16,), c, jnp.float32)


_BN = NP
_GRID = NP // _BN


def _prep0(xpad, w, asrc, adst, we, ae):
    rep = lambda shp: pl.BlockSpec(shp, lambda i: tuple(0 for _ in shp))
    return pl.pallas_call(
        _prep0_body,
        grid=(_GRID,),
        in_specs=[
            pl.BlockSpec((_BN, D), lambda i: (i, 0)),
            rep((D, H)), rep((1, H)), rep((1, H)), rep((1, H)), rep((1, H)),
        ],
        out_specs=[
            pl.BlockSpec((_BN, H), lambda i: (i, 0)),
            pl.BlockSpec((_BN,), lambda i: (i,)),
            pl.BlockSpec((_BN,), lambda i: (i,)),
            pl.BlockSpec((16,), lambda i: (0,)),
        ],
        out_shape=[
            jax.ShapeDtypeStruct((NP, H), jnp.float32),
            jax.ShapeDtypeStruct((NP,), jnp.float32),
            jax.ShapeDtypeStruct((NP,), jnp.float32),
            jax.ShapeDtypeStruct((16,), jnp.float32),
        ],
    )(xpad, w, asrc, adst, we, ae)


def _prep_next(acc, den, b, w, asrc, adst, we, ae, relu):
    rep = lambda shp: pl.BlockSpec(shp, lambda i: tuple(0 for _ in shp))
    return pl.pallas_call(
        functools.partial(_prep_next_body, relu=relu),
        grid=(_GRID,),
        in_specs=[
            pl.BlockSpec((2, _BN, H), lambda i: (0, i, 0)),
            pl.BlockSpec((2, _BN), lambda i: (0, i)),
            rep((1, H)),
            pl.BlockSpec((H, H), lambda i: (0, 0)),
            rep((1, H)), rep((1, H)), rep((1, H)), rep((1, H)),
        ],
        out_specs=[
            pl.BlockSpec((_BN, H), lambda i: (i, 0)),
            pl.BlockSpec((_BN,), lambda i: (i,)),
            pl.BlockSpec((_BN,), lambda i: (i,)),
            pl.BlockSpec((16,), lambda i: (0,)),
        ],
        out_shape=[
            jax.ShapeDtypeStruct((NP, H), jnp.float32),
            jax.ShapeDtypeStruct((NP,), jnp.float32),
            jax.ShapeDtypeStruct((NP,), jnp.float32),
            jax.ShapeDtypeStruct((16,), jnp.float32),
        ],
    )(acc, den, b, w, asrc, adst, we, ae)


def _gat_edge_body(src_hbm, dst_hbm, ea_hbm, wx_hbm, ss_hbm, sd_hbm, cv_hbm,
                   den_out, acc_out, *rest):
    bufs = []
    for b in range(NBUF):
        bufs.append(tuple(rest[b * 5:(b + 1) * 5]))
    (srcres, dstres, eares, cvv, stage, dstage,
     den_sp, acc_sp) = rest[NBUF * 5:NBUF * 5 + 8]
    semL = rest[NBUF * 5 + 8]
    sems = rest[NBUF * 5 + 9:]
    for b in range(NBUF):
        bufs[b] = bufs[b] + tuple(sems[b * 2:(b + 1) * 2])

    cid = lax.axis_index("c")
    sid = lax.axis_index("s")
    wid = sid * NC + cid

    pltpu.async_copy(src_hbm.at[wid], srcres, semL)
    pltpu.async_copy(dst_hbm.at[wid], dstres, semL)
    pltpu.async_copy(ea_hbm.at[wid], eares, semL)
    pltpu.sync_copy(cv_hbm, cvv)

    z16 = jnp.zeros((16,), jnp.float32)

    def _zrow(r, _):
        stage[r, pl.ds(0, 16)] = z16
        stage[r, pl.ds(16, 16)] = z16
        return 0
    lax.fori_loop(0, NODES_PER_TILE, _zrow, 0)

    def _zden(j, _):
        dstage[pl.ds(j * 16, 16)] = z16
        return 0
    lax.fori_loop(0, NODES_PER_TILE // 16, _zden, 0)

    nbase = sid * NODES_PER_TILE
    pltpu.sync_copy(stage, acc_sp.at[pl.ds(nbase, NODES_PER_TILE), :])
    pltpu.sync_copy(dstage, den_sp.at[pl.ds(nbase, NODES_PER_TILE)])
    plsc.subcore_barrier()

    pltpu.make_async_copy(src_hbm.at[0], srcres, semL).wait()
    pltpu.make_async_copy(dst_hbm.at[0], dstres, semL).wait()
    pltpu.make_async_copy(ea_hbm.at[0], eares, semL).wait()

    def issue_g(buf, k):
        _, ssg, sdg, rows, _, semG, _ = buf
        pltpu.async_copy(ss_hbm.at[srcres.at[k]], ssg, semG)
        pltpu.async_copy(sd_hbm.at[dstres.at[k]], sdg, semG)
        pltpu.async_copy(wx_hbm.at[srcres.at[k]], rows, semG)

    def wait_g(buf):
        _, ssg, sdg, rows, _, semG, _ = buf
        pltpu.make_async_copy(ss_hbm.at[srcres.at[0]], ssg, semG).wait()
        pltpu.make_async_copy(sd_hbm.at[dstres.at[0]], sdg, semG).wait()
        pltpu.make_async_copy(wx_hbm.at[srcres.at[0]], rows, semG).wait()

    def compute_core(buf, dst_ld, ea_ld):
        dstv_s, ssg, sdg, rows, av, _, _ = buf
        cvec = cvv[...]
        for j in range(CH // 16):
            dstv_s[pl.ds(j * 16, 16)] = dst_ld(j)
            s = ssg[pl.ds(j * 16, 16)] + sdg[pl.ds(j * 16, 16)] \
                + cvec * ea_ld(j)
            s = jnp.where(s > 0.0, s, 0.2 * s)
            av[pl.ds(j * 16, 16)] = jnp.exp(s)
        for j in range(CH // 16):
            a16 = av[pl.ds(j * 16, 16)]
            for i in range(16):
                e = j * 16 + i
                a = a16[i]
                rows[e, pl.ds(0, 16)] = rows[e, pl.ds(0, 16)] * a
                rows[e, pl.ds(16, 16)] = rows[e, pl.ds(16, 16)] * a

    def compute(buf, k):
        compute_core(buf,
                     lambda j: dstres[k, pl.ds(j * 16, 16)],
                     lambda j: eares[k, pl.ds(j * 16, 16)])

    def issue_s(buf):
        dstv_s, _, _, rows, av, _, semS = buf
        pltpu.async_copy(av, den_sp.at[dstv_s], semS, add=True)
        pltpu.async_copy(rows, acc_sp.at[dstv_s], semS, add=True)

    def wait_s(buf):
        dstv_s, _, _, rows, av, _, semS = buf
        pltpu.make_async_copy(av, den_sp.at[dstv_s], semS).wait()
        pltpu.make_async_copy(rows, acc_sp.at[dstv_s], semS).wait()

    for b in range(NBUF - 1):
        issue_g(bufs[b], b)

    def _body(i, _):
        for b in range(NBUF):
            k = i * NBUF + b
            buf = bufs[b]
            b3 = (b + NBUF - 1) % NBUF

            wait_g(buf)
            compute(buf, k)
            issue_s(buf)

            @pl.when(jnp.logical_and(k >= 1, k + NBUF - 1 < NCHT))
            def _():
                wait_s(bufs[b3])

            @pl.when(k + NBUF - 1 < NCHT)
            def _():
                issue_g(bufs[b3], k + NBUF - 1)
        return 0

    lax.fori_loop(0, NCHT // NBUF, _body, 0)
    for b in range(NBUF):
        wait_s(bufs[b])

    plsc.subcore_barrier()

    pltpu.sync_copy(acc_sp.at[pl.ds(nbase, NODES_PER_TILE), :], stage)
    pltpu.sync_copy(stage, acc_out.at[cid, pl.ds(nbase, NODES_PER_TILE), :])
    pltpu.sync_copy(den_sp.at[pl.ds(nbase, NODES_PER_TILE)], dstage)
    pltpu.sync_copy(dstage, den_out.at[cid, pl.ds(nbase, NODES_PER_TILE)])


_gat_edge = pl.kernel(
    _gat_edge_body,
    out_type=[
        jax.ShapeDtypeStruct((NC, NP), jnp.float32),
        jax.ShapeDtypeStruct((NC, NP, H), jnp.float32),
    ],
    mesh=_mesh,
    compiler_params=pltpu.CompilerParams(use_tc_tiling_on_sc=False),
    scratch_types=(
        [pltpu.VMEM((CH,), jnp.int32),
         pltpu.VMEM((CH,), jnp.float32),
         pltpu.VMEM((CH,), jnp.float32),
         pltpu.VMEM((CH, H), jnp.float32),
         pltpu.VMEM((CH,), jnp.float32)]
        * NBUF
        + [pltpu.VMEM((NCHT, CH), jnp.int32),
           pltpu.VMEM((NCHT, CH), jnp.int32),
           pltpu.VMEM((NCHT, CH), jnp.float32),
           pltpu.VMEM((16,), jnp.float32),
           pltpu.VMEM((NODES_PER_TILE, H), jnp.float32),
           pltpu.VMEM((NODES_PER_TILE,), jnp.float32),
           pltpu.VMEM_SHARED((NP,), jnp.float32),
           pltpu.VMEM_SHARED((NP, H), jnp.float32)]
        + [pltpu.SemaphoreType.DMA] * (1 + 2 * NBUF)
    ),
)


def _norm_final_body(acc_ref, den_ref, b_ref, h_ref):
    den = den_ref[0, :] + den_ref[1, :]
    h_ref[...] = (acc_ref[0, :, :] + acc_ref[1, :, :]) / (den[:, None] + 1e-16) \
        + b_ref[...]


def _norm_final(acc, den, b):
    return pl.pallas_call(
        _norm_final_body,
        grid=(_GRID,),
        in_specs=[
            pl.BlockSpec((2, _BN, H), lambda i: (0, i, 0)),
            pl.BlockSpec((2, _BN), lambda i: (0, i)),
            pl.BlockSpec((1, H), lambda i: (0, 0)),
        ],
        out_specs=pl.BlockSpec((_BN, H), lambda i: (i, 0)),
        out_shape=jax.ShapeDtypeStruct((NP, H), jnp.float32),
    )(acc, den, b)


def _emb_gather_body(h_hbm, cs_hbm, emb_out, idxv, rowsv, sem):
    cid = lax.axis_index("c")
    sid = lax.axis_index("s")
    wid = sid * NC + cid

    @pl.when(wid < B // 8)
    def _():
        base = wid * 8
        pltpu.sync_copy(cs_hbm.at[pl.ds(base, 8)], idxv)
        pltpu.async_copy(h_hbm.at[idxv], rowsv, sem).wait()
        pltpu.sync_copy(rowsv, emb_out.at[pl.ds(base, 8), :])


_emb_gather = pl.kernel(
    _emb_gather_body,
    out_type=[jax.ShapeDtypeStruct((B, H), jnp.float32)],
    mesh=_mesh,
    compiler_params=pltpu.CompilerParams(use_tc_tiling_on_sc=False),
    scratch_types=[
        pltpu.VMEM((8,), jnp.int32),
        pltpu.VMEM((8, H), jnp.float32),
        pltpu.SemaphoreType.DMA,
    ],
)


def _gru_body(emb_ref, tgt_ref,
              wi0r_ref, wi0z_ref, wi0n_ref, wh0r_ref, wh0z_ref, wh0n_ref,
              wi1r_ref, wi1z_ref, wi1n_ref, wh1r_ref, wh1z_ref, wh1n_ref,
              bi0r_ref, bi0z_ref, bi0n_ref, bh0r_ref, bh0z_ref, bh0n_ref,
              bi1r_ref, bi1z_ref, bi1n_ref, bh1r_ref, bh1z_ref, bh1n_ref,
              wout_ref, bout_ref,
              y_ref, gir_s, giz_s, gin_s):
    tgt = tgt_ref[...]
    dot = lambda a, b: jnp.dot(a, b, preferred_element_type=jnp.float32)
    gir_s[...] = dot(tgt, wi0r_ref[...]) + bi0r_ref[...]
    giz_s[...] = dot(tgt, wi0z_ref[...]) + bi0z_ref[...]
    gin_s[...] = dot(tgt, wi0n_ref[...]) + bi0n_ref[...]

    emb = emb_ref[...]

    def step(t, carry):
        h0, h1 = carry
        r0 = jax.nn.sigmoid(gir_s[pl.ds(t * B, B), :]
                            + dot(h0, wh0r_ref[...]) + bh0r_ref[...])
        z0 = jax.nn.sigmoid(giz_s[pl.ds(t * B, B), :]
                            + dot(h0, wh0z_ref[...]) + bh0z_ref[...])
        n0 = jnp.tanh(gin_s[pl.ds(t * B, B), :]
                      + r0 * (dot(h0, wh0n_ref[...]) + bh0n_ref[...]))
        h0n = (1.0 - z0) * n0 + z0 * h0

        r1 = jax.nn.sigmoid(dot(h0n, wi1r_ref[...]) + bi1r_ref[...]
                            + dot(h1, wh1r_ref[...]) + bh1r_ref[...])
        z1 = jax.nn.sigmoid(dot(h0n, wi1z_ref[...]) + bi1z_ref[...]
                            + dot(h1, wh1z_ref[...]) + bh1z_ref[...])
        n1 = jnp.tanh(dot(h0n, wi1n_ref[...]) + bi1n_ref[...]
                      + r1 * (dot(h1, wh1n_ref[...]) + bh1n_ref[...]))
        h1n = (1.0 - z1) * n1 + z1 * h1

        y_ref[pl.ds(t * B, B), :] = dot(h1n, wout_ref[...]) + bout_ref[...]
        return (h0n, h1n)

    lax.fori_loop(0, T, step, (emb, emb))


def _gru(emb, tgt_tmaj, *args):
    return pl.pallas_call(
        _gru_body,
        out_shape=jax.ShapeDtypeStruct((T * B, D), jnp.float32),
        scratch_shapes=[
            pltpu.VMEM((T * B, H), jnp.float32),
            pltpu.VMEM((T * B, H), jnp.float32),
            pltpu.VMEM((T * B, H), jnp.float32),
        ],
    )(emb, tgt_tmaj, *args)


def kernel(x, edge_index, edge_attr, cur_states, tgt_seq,
           gat0_W, gat0_We, gat0_asrc, gat0_adst, gat0_ae, gat0_b,
           gat1_W, gat1_We, gat1_asrc, gat1_adst, gat1_ae, gat1_b,
           gat2_W, gat2_We, gat2_asrc, gat2_adst, gat2_ae, gat2_b,
           W_ih0, W_hh0, b_ih0, b_hh0,
           W_ih1, W_hh1, b_ih1, b_hh1,
           W_out, b_out):
    xpad = jnp.pad(x, ((0, NP - N), (0, 0)))
    pad_idx = N + (jnp.arange(EPAD, dtype=jnp.int32) % (NP - N))
    src2d = jnp.concatenate([edge_index[0], pad_idx]).reshape(NW, NCHT, CH)
    dst2d = jnp.concatenate([edge_index[1], pad_idx]).reshape(NW, NCHT, CH)
    ea2d = jnp.concatenate([edge_attr[:, 0],
                            jnp.zeros((EPAD,), jnp.float32)]).reshape(NW, NCHT, CH)

    r1 = lambda v: v.reshape(1, -1)

    wx, ss, sd, cv = _prep0(xpad, gat0_W, r1(gat0_asrc), r1(gat0_adst),
                            gat0_We, r1(gat0_ae))
    den, acc = _gat_edge(src2d, dst2d, ea2d, wx, ss, sd, cv)

    wx, ss, sd, cv = _prep_next(acc, den, r1(gat0_b), gat1_W, r1(gat1_asrc),
                                r1(gat1_adst), gat1_We, r1(gat1_ae), relu=True)
    den, acc = _gat_edge(src2d, dst2d, ea2d, wx, ss, sd, cv)

    wx, ss, sd, cv = _prep_next(acc, den, r1(gat1_b), gat2_W, r1(gat2_asrc),
                                r1(gat2_adst), gat2_We, r1(gat2_ae), relu=True)
    den, acc = _gat_edge(src2d, dst2d, ea2d, wx, ss, sd, cv)

    h2 = _norm_final(acc, den, r1(gat2_b))
    (emb,) = _emb_gather(h2, cur_states)

    tgt_tmaj = jnp.swapaxes(tgt_seq, 0, 1).reshape(T * B, D)
    wi0 = W_ih0.T
    wh0 = W_hh0.T
    wi1 = W_ih1.T
    wh1 = W_hh1.T
    y = _gru(emb, tgt_tmaj,
             wi0[:, 0:H], wi0[:, H:2 * H], wi0[:, 2 * H:3 * H],
             wh0[:, 0:H], wh0[:, H:2 * H], wh0[:, 2 * H:3 * H],
             wi1[:, 0:H], wi1[:, H:2 * H], wi1[:, 2 * H:3 * H],
             wh1[:, 0:H], wh1[:, H:2 * H], wh1[:, 2 * H:3 * H],
             r1(b_ih0[0:H]), r1(b_ih0[H:2 * H]), r1(b_ih0[2 * H:3 * H]),
             r1(b_hh0[0:H]), r1(b_hh0[H:2 * H]), r1(b_hh0[2 * H:3 * H]),
             r1(b_ih1[0:H]), r1(b_ih1[H:2 * H]), r1(b_ih1[2 * H:3 * H]),
             r1(b_hh1[0:H]), r1(b_hh1[H:2 * H]), r1(b_hh1[2 * H:3 * H]),
             W_out.T, r1(b_out))
    y = y.reshape(T, B, D).swapaxes(0, 1)
    return y
